# baseline (device time: 184167 ns/iter reference)
import jax
import jax.numpy as jnp
from jax import lax
from jax.experimental import pallas as pl
from jax.experimental.pallas import tpu as pltpu

N_DEV = 8


def _gelu(y):
    c = 0.7978845608028654
    return 0.5 * y * (1.0 + jnp.tanh(c * (y + 0.044715 * y * y * y)))


def kernel(x, w_mat):
    m_per, k = x.shape
    _, n_per = w_mat.shape

    def body(x_ref, w_ref, out_ref, comm_ref, send_sems, recv_sems):
        my = lax.axis_index("i")
        left = lax.rem(my - 1 + N_DEV, N_DEV)
        right = lax.rem(my + 1, N_DEV)

        barrier_sem = pltpu.get_barrier_semaphore()
        for nbr in [left, right]:
            pl.semaphore_signal(
                barrier_sem, inc=1,
                device_id=(nbr,), device_id_type=pl.DeviceIdType.MESH,
            )
        pl.semaphore_wait(barrier_sem, 2)

        w = w_ref[:, :]

        out_ref[pl.ds(my * m_per, m_per), :] = _gelu(
            jnp.dot(x_ref[:, :], w, preferred_element_type=jnp.float32)
        )

        for h in range(N_DEV - 1):
            src = x_ref if h == 0 else comm_ref.at[h - 1]
            rdma = pltpu.make_async_remote_copy(
                src_ref=src,
                dst_ref=comm_ref.at[h],
                send_sem=send_sems.at[h],
                recv_sem=recv_sems.at[h],
                device_id=(right,),
                device_id_type=pl.DeviceIdType.MESH,
            )
            rdma.start()
            rdma.wait()

            origin = lax.rem(my - h - 1 + N_DEV, N_DEV)
            out_ref[pl.ds(origin * m_per, m_per), :] = _gelu(
                jnp.dot(comm_ref[h], w, preferred_element_type=jnp.float32)
            )

    return pl.pallas_call(
        body,
        out_shape=jax.ShapeDtypeStruct((N_DEV * m_per, n_per), jnp.float32),
        in_specs=[
            pl.BlockSpec(memory_space=pltpu.VMEM),
            pl.BlockSpec(memory_space=pltpu.VMEM),
        ],
        out_specs=pl.BlockSpec(memory_space=pltpu.VMEM),
        scratch_shapes=[
            pltpu.VMEM((N_DEV - 1, m_per, k), jnp.float32),
            pltpu.SemaphoreType.DMA((N_DEV - 1,)),
            pltpu.SemaphoreType.DMA((N_DEV - 1,)),
        ],
        compiler_params=pltpu.CompilerParams(collective_id=0),
    )(x, w_mat)


# device time: 96842 ns/iter; 1.9017x vs baseline; 1.9017x over previous
import jax
import jax.numpy as jnp
from jax import lax
from jax.experimental import pallas as pl
from jax.experimental.pallas import tpu as pltpu

N_DEV = 8
N_HOP = N_DEV // 2


def _gelu(y):
    c = 0.7978845608028654
    return 0.5 * y * (1.0 + jnp.tanh(c * (y + 0.044715 * y * y * y)))


def kernel(x, w_mat):
    m_per, k = x.shape
    _, n_per = w_mat.shape
    half = m_per // 2

    def body(x_ref, w_ref, out_ref,
             comm_r, comm_l, send_r, recv_r, send_l, recv_l):
        my = lax.axis_index("i")
        left = lax.rem(my - 1 + N_DEV, N_DEV)
        right = lax.rem(my + 1, N_DEV)

        barrier_sem = pltpu.get_barrier_semaphore()
        for nbr in [left, right]:
            pl.semaphore_signal(
                barrier_sem, inc=1,
                device_id=(nbr,), device_id_type=pl.DeviceIdType.MESH,
            )
        pl.semaphore_wait(barrier_sem, 2)

        def hop_rdmas(h):
            if h == 0:
                src_r = x_ref
                src_l = x_ref
            else:
                src_r = comm_r.at[h - 1]
                src_l = comm_l.at[h - 1]
            if h == N_HOP - 1:
                src_r = src_r.at[pl.ds(0, half), :]
                src_l = src_l.at[pl.ds(half, half), :]
                dst_r = comm_r.at[h, pl.ds(0, half), :]
                dst_l = comm_l.at[h, pl.ds(half, half), :]
            else:
                dst_r = comm_r.at[h]
                dst_l = comm_l.at[h]
            r = pltpu.make_async_remote_copy(
                src_ref=src_r, dst_ref=dst_r,
                send_sem=send_r.at[h], recv_sem=recv_r.at[h],
                device_id=(right,), device_id_type=pl.DeviceIdType.MESH,
            )
            l = pltpu.make_async_remote_copy(
                src_ref=src_l, dst_ref=dst_l,
                send_sem=send_l.at[h], recv_sem=recv_l.at[h],
                device_id=(left,), device_id_type=pl.DeviceIdType.MESH,
            )
            return r, l

        w = w_ref[:, :]
        inflight = []

        r0, l0 = hop_rdmas(0)
        r0.start()
        l0.start()
        inflight += [r0, l0]
        out_ref[pl.ds(my * m_per, m_per), :] = _gelu(
            jnp.dot(x_ref[:, :], w, preferred_element_type=jnp.float32)
        )
        r0.wait_recv()
        l0.wait_recv()

        for h in range(1, N_HOP):
            rh, lh = hop_rdmas(h)
            rh.start()
            lh.start()
            inflight += [rh, lh]
            o_r = lax.rem(my - h + N_DEV, N_DEV)
            o_l = lax.rem(my + h, N_DEV)
            out_ref[pl.ds(o_r * m_per, m_per), :] = _gelu(
                jnp.dot(comm_r[h - 1], w, preferred_element_type=jnp.float32)
            )
            out_ref[pl.ds(o_l * m_per, m_per), :] = _gelu(
                jnp.dot(comm_l[h - 1], w, preferred_element_type=jnp.float32)
            )
            rh.wait_recv()
            lh.wait_recv()

        o_far = lax.rem(my + N_HOP, N_DEV)
        out_ref[pl.ds(o_far * m_per, half), :] = _gelu(
            jnp.dot(comm_r[N_HOP - 1, :half, :], w,
                    preferred_element_type=jnp.float32)
        )
        out_ref[pl.ds(o_far * m_per + half, half), :] = _gelu(
            jnp.dot(comm_l[N_HOP - 1, half:, :], w,
                    preferred_element_type=jnp.float32)
        )

        for rdma in inflight:
            rdma.wait_send()

    return pl.pallas_call(
        body,
        out_shape=jax.ShapeDtypeStruct((N_DEV * m_per, n_per), jnp.float32),
        in_specs=[
            pl.BlockSpec(memory_space=pltpu.VMEM),
            pl.BlockSpec(memory_space=pltpu.VMEM),
        ],
        out_specs=pl.BlockSpec(memory_space=pltpu.VMEM),
        scratch_shapes=[
            pltpu.VMEM((N_HOP, m_per, k), jnp.float32),
            pltpu.VMEM((N_HOP, m_per, k), jnp.float32),
            pltpu.SemaphoreType.DMA((N_HOP,)),
            pltpu.SemaphoreType.DMA((N_HOP,)),
            pltpu.SemaphoreType.DMA((N_HOP,)),
            pltpu.SemaphoreType.DMA((N_HOP,)),
        ],
        compiler_params=pltpu.CompilerParams(collective_id=0),
    )(x, w_mat)


# device time: 90819 ns/iter; 2.0278x vs baseline; 1.0663x over previous
import jax
import jax.numpy as jnp
from jax import lax
from jax.experimental import pallas as pl
from jax.experimental.pallas import tpu as pltpu

N_DEV = 8
N_HOP = N_DEV // 2


def _gelu(y):
    c = 0.7978845608028654
    return 0.5 * y * (1.0 + jnp.tanh(c * (y + 0.044715 * y * y * y)))


def kernel(x, w_mat):
    m_per, k = x.shape
    _, n_per = w_mat.shape
    half = m_per // 2

    def body(x_ref, w_ref, out_ref,
             comm_r, comm_l, send_r, recv_r, send_l, recv_l):
        my = lax.axis_index("i")
        left = lax.rem(my - 1 + N_DEV, N_DEV)
        right = lax.rem(my + 1, N_DEV)

        barrier_sem = pltpu.get_barrier_semaphore()
        for nbr in [left, right]:
            pl.semaphore_signal(
                barrier_sem, inc=1,
                device_id=(nbr,), device_id_type=pl.DeviceIdType.MESH,
            )
        pl.semaphore_wait(barrier_sem, 2)

        def piece(h, p, go_right):
            rows = pl.ds(p * half, half)
            if h == 0:
                src = x_ref.at[rows, :]
            elif go_right:
                src = comm_r.at[h - 1, rows, :]
            else:
                src = comm_l.at[h - 1, rows, :]
            if go_right:
                return pltpu.make_async_remote_copy(
                    src_ref=src, dst_ref=comm_r.at[h, rows, :],
                    send_sem=send_r.at[h, p], recv_sem=recv_r.at[h, p],
                    device_id=(right,), device_id_type=pl.DeviceIdType.MESH,
                )
            return pltpu.make_async_remote_copy(
                src_ref=src, dst_ref=comm_l.at[h, rows, :],
                send_sem=send_l.at[h, p], recv_sem=recv_l.at[h, p],
                device_id=(left,), device_id_type=pl.DeviceIdType.MESH,
            )

        w = w_ref[:, :]
        inflight = []

        def start(rdma):
            rdma.start()
            inflight.append(rdma)
            return rdma

        r0a = start(piece(0, 0, True))
        l0b = start(piece(0, 1, False))
        r0b = start(piece(0, 1, True))
        l0a = start(piece(0, 0, False))
        out_ref[pl.ds(my * m_per, m_per), :] = _gelu(
            jnp.dot(x_ref[:, :], w, preferred_element_type=jnp.float32)
        )

        prev = {"ra": r0a, "rb": r0b, "la": l0a, "lb": l0b}
        for h in range(1, N_HOP):
            last = h == N_HOP - 1
            prev["ra"].wait_recv()
            ra = start(piece(h, 0, True))
            prev["lb"].wait_recv()
            lb = start(piece(h, 1, False))
            prev["rb"].wait_recv()
            prev["la"].wait_recv()
            rb = la = None
            if not last:
                rb = start(piece(h, 1, True))
                la = start(piece(h, 0, False))
            o_r = lax.rem(my - h + N_DEV, N_DEV)
            o_l = lax.rem(my + h, N_DEV)
            out_ref[pl.ds(o_r * m_per, m_per), :] = _gelu(
                jnp.dot(comm_r[h - 1], w, preferred_element_type=jnp.float32)
            )
            out_ref[pl.ds(o_l * m_per, m_per), :] = _gelu(
                jnp.dot(comm_l[h - 1], w, preferred_element_type=jnp.float32)
            )
            prev = {"ra": ra, "rb": rb, "la": la, "lb": lb}

        o_far = lax.rem(my + N_HOP, N_DEV)
        prev["ra"].wait_recv()
        out_ref[pl.ds(o_far * m_per, half), :] = _gelu(
            jnp.dot(comm_r[N_HOP - 1, :half, :], w,
                    preferred_element_type=jnp.float32)
        )
        prev["lb"].wait_recv()
        out_ref[pl.ds(o_far * m_per + half, half), :] = _gelu(
            jnp.dot(comm_l[N_HOP - 1, half:, :], w,
                    preferred_element_type=jnp.float32)
        )

        for rdma in inflight:
            rdma.wait_send()

    return pl.pallas_call(
        body,
        out_shape=jax.ShapeDtypeStruct((N_DEV * m_per, n_per), jnp.float32),
        in_specs=[
            pl.BlockSpec(memory_space=pltpu.VMEM),
            pl.BlockSpec(memory_space=pltpu.VMEM),
        ],
        out_specs=pl.BlockSpec(memory_space=pltpu.VMEM),
        scratch_shapes=[
            pltpu.VMEM((N_HOP, m_per, k), jnp.float32),
            pltpu.VMEM((N_HOP, m_per, k), jnp.float32),
            pltpu.SemaphoreType.DMA((N_HOP, 2)),
            pltpu.SemaphoreType.DMA((N_HOP, 2)),
            pltpu.SemaphoreType.DMA((N_HOP, 2)),
            pltpu.SemaphoreType.DMA((N_HOP, 2)),
        ],
        compiler_params=pltpu.CompilerParams(collective_id=0),
    )(x, w_mat)


# device time: 89797 ns/iter; 2.0509x vs baseline; 1.0114x over previous
import jax
import jax.numpy as jnp
import numpy as np
from jax import lax
from jax.experimental import pallas as pl
from jax.experimental.pallas import tpu as pltpu

N_DEV = 8
N_HOP = N_DEV // 2

_RING = np.array([0, 1, 2, 3, 7, 6, 5, 4])
_SUCC = np.empty(N_DEV, np.int32)
_PRED = np.empty(N_DEV, np.int32)
for _j in range(N_DEV):
    _SUCC[_RING[_j]] = _RING[(_j + 1) % N_DEV]
    _PRED[_RING[_j]] = _RING[(_j - 1) % N_DEV]
_PRED_POW = [np.arange(N_DEV)]
_SUCC_POW = [np.arange(N_DEV)]
for _j in range(N_HOP):
    _PRED_POW.append(_PRED[_PRED_POW[-1]])
    _SUCC_POW.append(_SUCC[_SUCC_POW[-1]])


def _lut(table, idx):
    r = jnp.int32(int(table[0]))
    for j in range(1, N_DEV):
        r = jnp.where(idx == j, jnp.int32(int(table[j])), r)
    return r


def _gelu(y):
    c = 0.7978845608028654
    return 0.5 * y * (1.0 + jnp.tanh(c * (y + 0.044715 * y * y * y)))


def kernel(x, w_mat):
    m_per, k = x.shape
    _, n_per = w_mat.shape
    half = m_per // 2

    def body(x_ref, w_ref, out_ref,
             comm_r, comm_l, send_r, recv_r, send_l, recv_l):
        my = lax.axis_index("i")
        left = _lut(_PRED, my)
        right = _lut(_SUCC, my)

        barrier_sem = pltpu.get_barrier_semaphore()
        for nbr in [left, right]:
            pl.semaphore_signal(
                barrier_sem, inc=1,
                device_id=(nbr,), device_id_type=pl.DeviceIdType.MESH,
            )
        pl.semaphore_wait(barrier_sem, 2)

        def piece(h, p, go_right):
            rows = pl.ds(p * half, half)
            if h == 0:
                src = x_ref.at[rows, :]
            elif go_right:
                src = comm_r.at[h - 1, rows, :]
            else:
                src = comm_l.at[h - 1, rows, :]
            if go_right:
                return pltpu.make_async_remote_copy(
                    src_ref=src, dst_ref=comm_r.at[h, rows, :],
                    send_sem=send_r.at[h, p], recv_sem=recv_r.at[h, p],
                    device_id=(right,), device_id_type=pl.DeviceIdType.MESH,
                )
            return pltpu.make_async_remote_copy(
                src_ref=src, dst_ref=comm_l.at[h, rows, :],
                send_sem=send_l.at[h, p], recv_sem=recv_l.at[h, p],
                device_id=(left,), device_id_type=pl.DeviceIdType.MESH,
            )

        w = w_ref[:, :]
        inflight = []

        def start(rdma):
            rdma.start()
            inflight.append(rdma)
            return rdma

        r0a = start(piece(0, 0, True))
        l0b = start(piece(0, 1, False))
        r0b = start(piece(0, 1, True))
        l0a = start(piece(0, 0, False))
        out_ref[pl.ds(my * m_per, m_per), :] = _gelu(
            jnp.dot(x_ref[:, :], w, preferred_element_type=jnp.float32)
        )

        prev = {"ra": r0a, "rb": r0b, "la": l0a, "lb": l0b}
        for h in range(1, N_HOP):
            last = h == N_HOP - 1
            prev["ra"].wait_recv()
            ra = start(piece(h, 0, True))
            prev["lb"].wait_recv()
            lb = start(piece(h, 1, False))
            prev["rb"].wait_recv()
            prev["la"].wait_recv()
            rb = la = None
            if not last:
                rb = start(piece(h, 1, True))
                la = start(piece(h, 0, False))
            o_r = _lut(_PRED_POW[h], my)
            o_l = _lut(_SUCC_POW[h], my)
            out_ref[pl.ds(o_r * m_per, m_per), :] = _gelu(
                jnp.dot(comm_r[h - 1], w, preferred_element_type=jnp.float32)
            )
            out_ref[pl.ds(o_l * m_per, m_per), :] = _gelu(
                jnp.dot(comm_l[h - 1], w, preferred_element_type=jnp.float32)
            )
            prev = {"ra": ra, "rb": rb, "la": la, "lb": lb}

        o_far = _lut(_PRED_POW[N_HOP], my)
        prev["ra"].wait_recv()
        out_ref[pl.ds(o_far * m_per, half), :] = _gelu(
            jnp.dot(comm_r[N_HOP - 1, :half, :], w,
                    preferred_element_type=jnp.float32)
        )
        prev["lb"].wait_recv()
        out_ref[pl.ds(o_far * m_per + half, half), :] = _gelu(
            jnp.dot(comm_l[N_HOP - 1, half:, :], w,
                    preferred_element_type=jnp.float32)
        )

        for rdma in inflight:
            rdma.wait_send()

    return pl.pallas_call(
        body,
        out_shape=jax.ShapeDtypeStruct((N_DEV * m_per, n_per), jnp.float32),
        in_specs=[
            pl.BlockSpec(memory_space=pltpu.VMEM),
            pl.BlockSpec(memory_space=pltpu.VMEM),
        ],
        out_specs=pl.BlockSpec(memory_space=pltpu.VMEM),
        scratch_shapes=[
            pltpu.VMEM((N_HOP, m_per, k), jnp.float32),
            pltpu.VMEM((N_HOP, m_per, k), jnp.float32),
            pltpu.SemaphoreType.DMA((N_HOP, 2)),
            pltpu.SemaphoreType.DMA((N_HOP, 2)),
            pltpu.SemaphoreType.DMA((N_HOP, 2)),
            pltpu.SemaphoreType.DMA((N_HOP, 2)),
        ],
        compiler_params=pltpu.CompilerParams(collective_id=0),
    )(x, w_mat)


# device time: 65858 ns/iter; 2.7964x vs baseline; 1.3635x over previous
import jax
import jax.numpy as jnp
import numpy as np
from jax import lax
from jax.experimental import pallas as pl
from jax.experimental.pallas import tpu as pltpu

N_DEV = 8

_NX = np.array([1, 0, 3, 2, 5, 4, 7, 6])
_NY = np.array([3, 2, 1, 0, 7, 6, 5, 4])
_NZ = np.array([4, 5, 6, 7, 0, 1, 2, 3])

_ORDERS = [(_NX, _NY, _NZ), (_NY, _NZ, _NX), (_NZ, _NX, _NY)]
_ROWS = [96, 80, 80]
_OFFS = [0, 96, 176]
N_STREAM = 3
N_MSG = 7


def _lut(table, idx):
    r = jnp.int32(int(table[0]))
    for j in range(1, N_DEV):
        r = jnp.where(idx == j, jnp.int32(int(table[j])), r)
    return r


def _gelu(y):
    c = 0.7978845608028654
    return 0.5 * y * (1.0 + jnp.tanh(c * (y + 0.044715 * y * y * y)))


def kernel(x, w_mat):
    m_per, k = x.shape
    _, n_per = w_mat.shape

    def body(x_ref, w_ref, out_ref, xg, ssem, rsem):
        my = lax.axis_index("i")

        nx = _lut(_NX, my)
        ny = _lut(_NY, my)
        nz = _lut(_NZ, my)
        barrier_sem = pltpu.get_barrier_semaphore()
        for nbr in [nx, ny, nz]:
            pl.semaphore_signal(
                barrier_sem, inc=1,
                device_id=(nbr,), device_id_type=pl.DeviceIdType.MESH,
            )
        pl.semaphore_wait(barrier_sem, 3)

        w = w_ref[:, :]
        inflight = []

        def piece_ref(s, origin):
            return xg.at[pl.ds(origin * m_per + _OFFS[s], _ROWS[s]), :]

        def send(s, j, src, origin, dev):
            rdma = pltpu.make_async_remote_copy(
                src_ref=src, dst_ref=piece_ref(s, origin),
                send_sem=ssem.at[s, j], recv_sem=rsem.at[s, j],
                device_id=(dev,), device_id_type=pl.DeviceIdType.MESH,
            )
            rdma.start()
            inflight.append(rdma)

        def wait_recv(s, j, origin):
            pltpu.make_async_remote_copy(
                src_ref=piece_ref(s, origin), dst_ref=piece_ref(s, origin),
                send_sem=ssem.at[s, j], recv_sem=rsem.at[s, j],
                device_id=(my,), device_id_type=pl.DeviceIdType.MESH,
            ).wait_recv()

        def gemm_piece(s, origin):
            rows = pl.ds(origin * m_per + _OFFS[s], _ROWS[s])
            out_ref[rows, :] = _gelu(
                jnp.dot(xg[rows, :], w, preferred_element_type=jnp.float32)
            )

        nbrs, orig = [], []
        for s in range(N_STREAM):
            d1, d2, d3 = _ORDERS[s]
            nbrs.append((_lut(d1, my), _lut(d2, my), _lut(d3, my)))
            orig.append({
                "o1": _lut(d1, my), "o2": _lut(d2, my),
                "o12": _lut(d1[d2], my), "o3": _lut(d3, my),
                "o13": _lut(d1[d3], my), "o23": _lut(d2[d3], my),
                "o123": _lut(d1[d2[d3]], my),
            })

        def own_src(s):
            return x_ref.at[pl.ds(_OFFS[s], _ROWS[s]), :]

        for s in range(N_STREAM):
            send(s, 0, own_src(s), my, nbrs[s][0])
        for s in range(N_STREAM):
            send(s, 1, own_src(s), my, nbrs[s][1])
        for s in range(N_STREAM):
            send(s, 2, own_src(s), my, nbrs[s][2])

        out_ref[pl.ds(my * m_per, m_per), :] = _gelu(
            jnp.dot(x_ref[:, :], w, preferred_element_type=jnp.float32)
        )

        for s in range(N_STREAM):
            o = orig[s]
            wait_recv(s, 0, o["o1"])
            send(s, 3, piece_ref(s, o["o1"]), o["o1"], nbrs[s][1])
            send(s, 4, piece_ref(s, o["o1"]), o["o1"], nbrs[s][2])
        for s in range(N_STREAM):
            gemm_piece(s, orig[s]["o1"])

        for s in range(N_STREAM):
            o = orig[s]
            wait_recv(s, 1, o["o2"])
            send(s, 5, piece_ref(s, o["o2"]), o["o2"], nbrs[s][2])
        for s in range(N_STREAM):
            o = orig[s]
            wait_recv(s, 3, o["o12"])
            send(s, 6, piece_ref(s, o["o12"]), o["o12"], nbrs[s][2])
        for s in range(N_STREAM):
            gemm_piece(s, orig[s]["o2"])
            gemm_piece(s, orig[s]["o12"])

        for key, j in [("o3", 2), ("o13", 4), ("o23", 5), ("o123", 6)]:
            for s in range(N_STREAM):
                wait_recv(s, j, orig[s][key])
            for s in range(N_STREAM):
                gemm_piece(s, orig[s][key])

        for rdma in inflight:
            rdma.wait_send()

    return pl.pallas_call(
        body,
        out_shape=jax.ShapeDtypeStruct((N_DEV * m_per, n_per), jnp.float32),
        in_specs=[
            pl.BlockSpec(memory_space=pltpu.VMEM),
            pl.BlockSpec(memory_space=pltpu.VMEM),
        ],
        out_specs=pl.BlockSpec(memory_space=pltpu.VMEM),
        scratch_shapes=[
            pltpu.VMEM((N_DEV * m_per, k), jnp.float32),
            pltpu.SemaphoreType.DMA((N_STREAM, N_MSG)),
            pltpu.SemaphoreType.DMA((N_STREAM, N_MSG)),
        ],
        compiler_params=pltpu.CompilerParams(collective_id=0),
    )(x, w_mat)


# device time: 64415 ns/iter; 2.8591x vs baseline; 1.0224x over previous
import jax
import jax.numpy as jnp
import numpy as np
from jax import lax
from jax.experimental import pallas as pl
from jax.experimental.pallas import tpu as pltpu

N_DEV = 8

_NX = np.array([1, 0, 3, 2, 5, 4, 7, 6])
_NY = np.array([3, 2, 1, 0, 7, 6, 5, 4])
_NZ = np.array([4, 5, 6, 7, 0, 1, 2, 3])

_ORDERS = [(_NX, _NY, _NZ), (_NY, _NZ, _NX), (_NZ, _NX, _NY)]
_ROWS = [88, 88, 80]
_OFFS = [0, 88, 176]
N_STREAM = 3
N_MSG = 7


def _lut(table, idx):
    r = jnp.int32(int(table[0]))
    for j in range(1, N_DEV):
        r = jnp.where(idx == j, jnp.int32(int(table[j])), r)
    return r


def _gelu(y):
    c = 0.7978845608028654
    return 0.5 * y * (1.0 + jnp.tanh(c * (y + 0.044715 * y * y * y)))


def kernel(x, w_mat):
    m_per, k = x.shape
    _, n_per = w_mat.shape

    def body(x_ref, w_ref, out_ref, xg, ssem, rsem):
        my = lax.axis_index("i")

        nx = _lut(_NX, my)
        ny = _lut(_NY, my)
        nz = _lut(_NZ, my)
        barrier_sem = pltpu.get_barrier_semaphore()
        for nbr in [nx, ny, nz]:
            pl.semaphore_signal(
                barrier_sem, inc=1,
                device_id=(nbr,), device_id_type=pl.DeviceIdType.MESH,
            )
        pl.semaphore_wait(barrier_sem, 3)

        w = w_ref[:, :]
        inflight = []

        def piece_ref(s, origin):
            return xg.at[pl.ds(origin * m_per + _OFFS[s], _ROWS[s]), :]

        def send(s, j, src, origin, dev):
            rdma = pltpu.make_async_remote_copy(
                src_ref=src, dst_ref=piece_ref(s, origin),
                send_sem=ssem.at[s, j], recv_sem=rsem.at[s, j],
                device_id=(dev,), device_id_type=pl.DeviceIdType.MESH,
            )
            rdma.start()
            inflight.append(rdma)

        def wait_recv(s, j, origin):
            pltpu.make_async_remote_copy(
                src_ref=piece_ref(s, origin), dst_ref=piece_ref(s, origin),
                send_sem=ssem.at[s, j], recv_sem=rsem.at[s, j],
                device_id=(my,), device_id_type=pl.DeviceIdType.MESH,
            ).wait_recv()

        def gemm_piece(s, origin):
            rows = pl.ds(origin * m_per + _OFFS[s], _ROWS[s])
            out_ref[rows, :] = _gelu(
                jnp.dot(xg[rows, :], w, preferred_element_type=jnp.float32)
            )

        nbrs, orig = [], []
        for s in range(N_STREAM):
            d1, d2, d3 = _ORDERS[s]
            nbrs.append((_lut(d1, my), _lut(d2, my), _lut(d3, my)))
            orig.append({
                "o1": _lut(d1, my), "o2": _lut(d2, my),
                "o12": _lut(d1[d2], my), "o3": _lut(d3, my),
                "o13": _lut(d1[d3], my), "o23": _lut(d2[d3], my),
                "o123": _lut(d1[d2[d3]], my),
            })

        def own_src(s):
            return x_ref.at[pl.ds(_OFFS[s], _ROWS[s]), :]

        for s in range(N_STREAM):
            send(s, 0, own_src(s), my, nbrs[s][0])
        for s in range(N_STREAM):
            send(s, 1, own_src(s), my, nbrs[s][1])
        for s in range(N_STREAM):
            send(s, 2, own_src(s), my, nbrs[s][2])

        out_ref[pl.ds(my * m_per, m_per), :] = _gelu(
            jnp.dot(x_ref[:, :], w, preferred_element_type=jnp.float32)
        )

        for s in range(N_STREAM):
            o = orig[s]
            wait_recv(s, 0, o["o1"])
            send(s, 3, piece_ref(s, o["o1"]), o["o1"], nbrs[s][1])
            send(s, 4, piece_ref(s, o["o1"]), o["o1"], nbrs[s][2])
        for s in range(N_STREAM):
            gemm_piece(s, orig[s]["o1"])

        for s in range(N_STREAM):
            o = orig[s]
            wait_recv(s, 1, o["o2"])
            send(s, 5, piece_ref(s, o["o2"]), o["o2"], nbrs[s][2])
        for s in range(N_STREAM):
            o = orig[s]
            wait_recv(s, 3, o["o12"])
            send(s, 6, piece_ref(s, o["o12"]), o["o12"], nbrs[s][2])
        for s in range(N_STREAM):
            gemm_piece(s, orig[s]["o2"])
            gemm_piece(s, orig[s]["o12"])

        for key, j in [("o3", 2), ("o13", 4), ("o23", 5), ("o123", 6)]:
            for s in range(N_STREAM):
                wait_recv(s, j, orig[s][key])
            for s in range(N_STREAM):
                gemm_piece(s, orig[s][key])

        for rdma in inflight:
            rdma.wait_send()

    return pl.pallas_call(
        body,
        out_shape=jax.ShapeDtypeStruct((N_DEV * m_per, n_per), jnp.float32),
        in_specs=[
            pl.BlockSpec(memory_space=pltpu.VMEM),
            pl.BlockSpec(memory_space=pltpu.VMEM),
        ],
        out_specs=pl.BlockSpec(memory_space=pltpu.VMEM),
        scratch_shapes=[
            pltpu.VMEM((N_DEV * m_per, k), jnp.float32),
            pltpu.SemaphoreType.DMA((N_STREAM, N_MSG)),
            pltpu.SemaphoreType.DMA((N_STREAM, N_MSG)),
        ],
        compiler_params=pltpu.CompilerParams(collective_id=0),
    )(x, w_mat)
